# Initial kernel scaffold; baseline (speedup 1.0000x reference)
import jax
import jax.numpy as jnp
from jax.experimental import pallas as pl


def kernel(x, Wg, w1, w2, w3):
    raise NotImplementedError("write your pallas kernel here")


# speedup vs baseline: 1.1479x; 1.1479x over previous
import functools

import jax
import jax.numpy as jnp
from jax.experimental import pallas as pl
from jax.experimental.pallas import tpu as pltpu

E = 8
K = 2
D = 768
F = 2048
ALPHA = 0.1


def _router_kernel(x_ref, wg_ref, cw_ref, aux_ref, ce_acc, ss_acc, *, T):
    i = pl.program_id(0)
    nt = pl.num_programs(0)
    x = x_ref[...]
    logits = jax.lax.dot_general(
        x, wg_ref[...], (((1,), (1,)), ((), ())),
        preferred_element_type=jnp.float32)
    m = jnp.max(logits, axis=1, keepdims=True)
    ex = jnp.exp(logits - m)
    scores = ex / jnp.sum(ex, axis=1, keepdims=True)

    lane = jax.lax.broadcasted_iota(jnp.int32, scores.shape, 1)
    s1 = jnp.max(scores, axis=1, keepdims=True)
    i1 = jnp.min(jnp.where(scores == s1, lane, E), axis=1, keepdims=True)
    masked = jnp.where(lane == i1, -jnp.inf, scores)
    s2 = jnp.max(masked, axis=1, keepdims=True)
    i2 = jnp.min(jnp.where(masked == s2, lane, E), axis=1, keepdims=True)
    denom = s1 + s2 + 1e-20
    oh1 = lane == i1
    oh2 = lane == i2
    cw_ref[...] = jnp.where(oh1, s1 / denom, 0.0) + jnp.where(oh2, s2 / denom, 0.0)

    @pl.when(i == 0)
    def _():
        ce_acc[...] = jnp.zeros_like(ce_acc)
        ss_acc[...] = jnp.zeros_like(ss_acc)

    ce_acc[...] += jnp.sum(
        oh1.astype(jnp.float32) + oh2.astype(jnp.float32), axis=0, keepdims=True)
    ss_acc[...] += jnp.sum(scores, axis=0, keepdims=True)

    @pl.when(i == nt - 1)
    def _():
        ce = ce_acc[...] / (T * K / E)
        aux_ref[...] = jnp.sum(ce * (ss_acc[...] / T), keepdims=True).reshape(1, 1) * ALPHA


def _ffn_kernel(x_ref, w1_ref, w3_ref, w2_ref, cw_ref, y_ref, y_acc, *, TM):
    e = pl.program_id(0)
    ft = pl.program_id(1)
    mi = pl.program_id(2)
    nft = pl.num_programs(1)
    nm = pl.num_programs(2)

    @pl.when((e == 0) & (ft == 0))
    def _():
        y_acc[pl.ds(mi * TM, TM), :] = jnp.zeros((TM, D), jnp.float32)

    x = x_ref[...]
    h1 = jax.lax.dot_general(
        x, w1_ref[0], (((1,), (1,)), ((), ())), preferred_element_type=jnp.float32)
    h3 = jax.lax.dot_general(
        x, w3_ref[0], (((1,), (1,)), ((), ())), preferred_element_type=jnp.float32)
    act = (h1 * jax.nn.sigmoid(h1) * h3).astype(x.dtype)
    eo = jax.lax.dot_general(
        act, w2_ref[0], (((1,), (1,)), ((), ())), preferred_element_type=jnp.float32)
    lane = jax.lax.broadcasted_iota(jnp.int32, cw_ref.shape, 1)
    col = jnp.sum(jnp.where(lane == e, cw_ref[...], 0.0), axis=1, keepdims=True)
    y_acc[pl.ds(mi * TM, TM), :] += col * eo

    @pl.when((e == E - 1) & (ft == nft - 1) & (mi == nm - 1))
    def _():
        y_ref[...] = y_acc[...]


def kernel(x, Wg, w1, w2, w3):
    bsz, seq_len, _ = x.shape
    T = bsz * seq_len
    xf = x.reshape(T, D)

    TM_R = 256
    cw, aux = pl.pallas_call(
        functools.partial(_router_kernel, T=T),
        grid=(T // TM_R,),
        in_specs=[
            pl.BlockSpec((TM_R, D), lambda i: (i, 0)),
            pl.BlockSpec((E, D), lambda i: (0, 0)),
        ],
        out_specs=[
            pl.BlockSpec((TM_R, E), lambda i: (i, 0)),
            pl.BlockSpec((1, 1), lambda i: (0, 0)),
        ],
        out_shape=[
            jax.ShapeDtypeStruct((T, E), jnp.float32),
            jax.ShapeDtypeStruct((1, 1), jnp.float32),
        ],
        scratch_shapes=[
            pltpu.VMEM((1, E), jnp.float32),
            pltpu.VMEM((1, E), jnp.float32),
        ],
    )(xf, Wg)

    TM = 256
    FT = 1024
    y = pl.pallas_call(
        functools.partial(_ffn_kernel, TM=TM),
        grid=(E, F // FT, T // TM),
        in_specs=[
            pl.BlockSpec((TM, D), lambda e, ft, mi: (mi, 0)),
            pl.BlockSpec((1, FT, D), lambda e, ft, mi: (e, ft, 0)),
            pl.BlockSpec((1, FT, D), lambda e, ft, mi: (e, ft, 0)),
            pl.BlockSpec((1, D, FT), lambda e, ft, mi: (e, 0, ft)),
            pl.BlockSpec((TM, E), lambda e, ft, mi: (mi, 0)),
        ],
        out_specs=pl.BlockSpec((T, D), lambda e, ft, mi: (0, 0)),
        out_shape=jax.ShapeDtypeStruct((T, D), jnp.float32),
        scratch_shapes=[pltpu.VMEM((T, D), jnp.float32)],
    )(xf, w1, w3, w2, cw)

    return y.reshape(bsz, seq_len, D), aux[0, 0]
